# baseline (speedup 1.0000x reference)
import functools
import math

import jax
import jax.numpy as jnp
from jax import lax
from jax.experimental import pallas as pl
from jax.experimental.pallas import tpu as pltpu
from jax.experimental.pallas import tpu_sc as plsc

_MARGIN = 0

_CW = 2944
_POS = _CW // 16


def _sc_stream(x, labels_pad, indexes):
    b, n = x.shape
    n_pad = labels_pad.shape[0]
    assert n_pad % _CW == 0
    n_chunks = n_pad // _CW
    info = plsc.get_sparse_core_info()
    nc = info.num_cores
    nw = info.num_cores * info.num_subcores
    rpw = b // nw
    n_groups = rpw // 8

    mesh = plsc.VectorSubcoreMesh(core_axis_name="c", subcore_axis_name="s")

    @functools.partial(
        pl.kernel,
        mesh=mesh,
        out_type=(
            jax.ShapeDtypeStruct((b * 16,), jnp.float32),
            jax.ShapeDtypeStruct((b * 16,), jnp.float32),
        ),
        scratch_types=[
            pltpu.VMEM((8, _CW), jnp.float32),
            pltpu.VMEM((8, _CW), jnp.float32),
            pltpu.VMEM((_CW,), jnp.int32),
            pltpu.VMEM((_CW,), jnp.int32),
            pltpu.VMEM((rpw,), jnp.int32),
            pltpu.VMEM((rpw,), jnp.int32),
            pltpu.VMEM((rpw * 16,), jnp.float32),
            pltpu.VMEM((rpw * 16,), jnp.float32),
            pltpu.SemaphoreType.DMA,
            pltpu.SemaphoreType.DMA,
            pltpu.SemaphoreType.DMA,
            pltpu.SemaphoreType.DMA,
        ],
    )
    def sc_kernel(x_hbm, lab_hbm, idx_hbm, p_hbm, z_hbm,
                  xb0, xb1, lb0, lb1, idx_v, y_v, p_acc, z_acc,
                  sem_x0, sem_x1, sem_l0, sem_l1):
        wid = lax.axis_index("s") * nc + lax.axis_index("c")
        row0 = wid * rpw
        pltpu.sync_copy(idx_hbm.at[pl.ds(row0, rpw)], idx_v)
        pltpu.async_copy(lab_hbm.at[idx_v], y_v, sem_l0).wait()

        iota = lax.iota(jnp.int32, 16)
        xbufs = (xb0, xb1)
        lbufs = (lb0, lb1)
        xsems = (sem_x0, sem_x1)
        lsems = (sem_l0, sem_l1)

        def do_group(g):
            grow = row0 + g * 8

            def start_dma(c, buf_i):
                pltpu.async_copy(
                    x_hbm.at[pl.ds(grow, 8), pl.ds(c * _CW, _CW)],
                    xbufs[buf_i], xsems[buf_i])
                pltpu.async_copy(
                    lab_hbm.at[pl.ds(c * _CW, _CW)],
                    lbufs[buf_i], lsems[buf_i])

            def wait_dma(buf_i):
                pltpu.make_async_copy(
                    x_hbm.at[pl.ds(grow, 8), pl.ds(0, _CW)],
                    xbufs[buf_i], xsems[buf_i]).wait()
                pltpu.make_async_copy(
                    lab_hbm.at[pl.ds(0, _CW)],
                    lbufs[buf_i], lsems[buf_i]).wait()

            yv16 = y_v[pl.ds((g // 2) * 16, 16)]
            iv16 = idx_v[pl.ds((g // 2) * 16, 16)]
            lo = (g % 2) * 8
            ys = [jnp.full((16,), yv16[lo + r], jnp.int32) for r in range(8)]
            ix = [jnp.full((16,), iv16[lo + r], jnp.int32) for r in range(8)]

            def do_chunk(c, buf_i):
                wait_dma(buf_i)
                xb = xbufs[buf_i]
                lb = lbufs[buf_i]
                accs0 = tuple(jnp.zeros((16,), jnp.float32)
                              for _ in range(16))

                def pos_step(i, carry):
                    accs = carry
                    colv = c * _CW + i * 16 + iota
                    lv = lb[pl.ds(i * 16, 16)]
                    valid = lv >= 0
                    out = []
                    for r in range(8):
                        az, ap = accs[2 * r], accs[2 * r + 1]
                        e = jnp.exp(xb[r, pl.ds(i * 16, 16)])
                        e = jnp.where((colv != ix[r]) & valid, e, 0.0)
                        az = az + e
                        ap = ap + jnp.where(lv == ys[r], e, 0.0)
                        out.extend((az, ap))
                    return tuple(out)

                accs = lax.fori_loop(0, _POS, pos_step, accs0)
                for r in range(8):
                    o = pl.ds((g * 8 + r) * 16, 16)
                    z_acc[o] = z_acc[o] + accs[2 * r]
                    p_acc[o] = p_acc[o] + accs[2 * r + 1]

            for r in range(8):
                o = pl.ds((g * 8 + r) * 16, 16)
                z_acc[o] = jnp.zeros((16,), jnp.float32)
                p_acc[o] = jnp.zeros((16,), jnp.float32)
            start_dma(0, 0)

            def two_chunks(t, _):
                c0 = t * 2

                @pl.when(c0 + 1 < n_chunks)
                def _():
                    start_dma(c0 + 1, 1)
                do_chunk(c0, 0)

                @pl.when(c0 + 1 < n_chunks)
                def _():
                    @pl.when(c0 + 2 < n_chunks)
                    def _():
                        start_dma(c0 + 2, 0)
                    do_chunk(c0 + 1, 1)
                return 0

            lax.fori_loop(0, (n_chunks + 1) // 2, two_chunks, 0)

            pltpu.sync_copy(
                p_acc.at[pl.ds(g * 128, 128)], p_hbm.at[pl.ds(grow * 16, 128)])
            pltpu.sync_copy(
                z_acc.at[pl.ds(g * 128, 128)], z_hbm.at[pl.ds(grow * 16, 128)])

        for g in range(n_groups):
            do_group(g)

    return sc_kernel(x, labels_pad, indexes)


def _tc_finalize(p16, z16):
    batch = p16.shape[0]
    out11 = jax.ShapeDtypeStruct((1, 1), jnp.float32)

    def body(p_ref, z_ref, loss_ref, min_ref, mean_ref):
        p = jnp.sum(p_ref[...], axis=1, keepdims=True)
        z = jnp.sum(z_ref[...], axis=1, keepdims=True)
        p = p * (1.0 / math.exp(_MARGIN))
        prob = p / z
        nzm = prob != 0.0
        logp = jnp.where(nzm, jnp.log(jnp.where(nzm, prob, 1.0)), 0.0)
        loss_ref[...] = jnp.full((1, 1), -1.0 / batch) * jnp.sum(logp)
        min_ref[...] = jnp.full((1, 1), 1.0) * jnp.min(p)
        mean_ref[...] = jnp.full((1, 1), 1.0 / batch) * jnp.sum(p)

    return pl.pallas_call(
        body,
        out_shape=[out11, out11, out11],
    )(p16, z16)


def kernel(x, features, labels, indexes):
    del features
    batch, n_cols = x.shape
    n_pad = _CW * ((n_cols + _CW - 1) // _CW)
    labels_pad = jnp.pad(labels, (0, n_pad - n_cols), constant_values=-1)
    p16, z16 = _sc_stream(x, labels_pad, indexes)
    loss, pmin, pmean = _tc_finalize(p16.reshape(batch, 16),
                                     z16.reshape(batch, 16))
    return (loss[0, 0], pmin[0, 0], pmean[0, 0])
